# baseline (speedup 1.0000x reference)
import jax
import jax.numpy as jnp
from jax import lax
from jax.experimental import pallas as pl
from jax.experimental.pallas import tpu as pltpu
from jax.experimental.pallas import tpu_sc as plsc

_VOCAB = 100000
_HID = 64
_MAXLEN = 200
_PAD_LEN = 208
_BATCH = 4096
_EPS = 1e-8
_NC = 2
_NS = 16
_NW = _NC * _NS
_SEQ_PER_W = _BATCH // _NW

_GATHER_DNUMS = lax.GatherDimensionNumbers(
    offset_dims=(), collapsed_slice_dims=(0,), start_index_map=(0,))


def _permute(x, idx):
    return lax.gather(x, idx[:, None], _GATHER_DNUMS, slice_sizes=(1,),
                      mode=lax.GatherScatterMode.PROMISE_IN_BOUNDS)


def _rsqrt(x):
    i = lax.bitcast_convert_type(x, jnp.int32)
    y = lax.bitcast_convert_type(jnp.int32(0x5F3759DF) - (i >> 1), jnp.float32)
    for _ in range(3):
        y = y * (1.5 - 0.5 * x * y * y)
    return y


def _body(tok_hbm, word_hbm, posT_hbm, gamma_hbm, beta_hbm, out_hbm,
          posT_v, tok_v, rows_v, xbuf_v, gamma_v, beta_v, sem):
    wid = lax.axis_index("s") * _NC + lax.axis_index("c")
    pltpu.sync_copy(posT_hbm, posT_v)
    pltpu.sync_copy(gamma_hbm, gamma_v)
    pltpu.sync_copy(beta_hbm, beta_v)
    g_regs = [gamma_v[pl.ds(16 * k, 16)] for k in range(4)]
    b_regs = [beta_v[pl.ds(16 * k, 16)] for k in range(4)]
    lane = lax.iota(jnp.int32, 16)

    def seq_body(s, carry):
        row = wid * _SEQ_PER_W + s
        pltpu.sync_copy(tok_hbm.at[pl.ds(row * _MAXLEN, _MAXLEN)],
                        tok_v.at[pl.ds(0, _MAXLEN)])
        c1 = pltpu.async_copy(word_hbm.at[tok_v.at[pl.ds(0, 96)]],
                              rows_v.at[pl.ds(0, 96)], sem)
        c2 = pltpu.async_copy(word_hbm.at[tok_v.at[pl.ds(96, 104)]],
                              rows_v.at[pl.ds(96, 104)], sem)
        c1.wait()
        c2.wait()

        def grp_body(g, c):
            t0 = g * 16
            tokidx = t0 + lane
            tokvec = tok_v[pl.ds(t0, 16)]
            maskf = jnp.where(tokvec != 0, jnp.float32(1.0), jnp.float32(0.0))
            ssum = None
            ssq = None
            for h in range(_HID):
                hconst = jnp.full((16,), h, jnp.int32)
                xh = plsc.load_gather(rows_v, [tokidx, hconst])
                xh = xh + posT_v[h, pl.ds(t0, 16)]
                xbuf_v[h, :] = xh
                sq = xh * xh
                ssum = xh if ssum is None else ssum + xh
                ssq = sq if ssq is None else ssq + sq
            mean = ssum * (1.0 / _HID)
            var = ssq * (1.0 / _HID) - mean * mean
            rinv = _rsqrt(var + _EPS)
            p = rinv * maskf
            for kk in range(4):
                gk = g_regs[kk]
                bk = b_regs[kk]
                for j in range(16):
                    h = kk * 16 + j
                    cj = jnp.full((16,), j, jnp.int32)
                    gs = _permute(gk, cj)
                    bs = _permute(bk, cj)
                    xh = xbuf_v[h, :]
                    out = (xh - mean) * (p * gs) + bs * maskf
                    hconst = jnp.full((16,), h, jnp.int32)
                    plsc.store_scatter(rows_v, [tokidx, hconst], out)
            return c

        lax.fori_loop(0, _PAD_LEN // 16, grp_body, 0)
        pltpu.sync_copy(rows_v.at[pl.ds(0, _MAXLEN)],
                        out_hbm.at[pl.ds(row * _MAXLEN, _MAXLEN)])
        return carry

    lax.fori_loop(0, _SEQ_PER_W, seq_body, 0)


_emb = pl.kernel(
    _body,
    mesh=plsc.VectorSubcoreMesh(core_axis_name="c", subcore_axis_name="s"),
    out_type=jax.ShapeDtypeStruct((_BATCH * _MAXLEN, _HID), jnp.float32),
    scratch_types=[
        pltpu.VMEM((_HID, _PAD_LEN), jnp.float32),
        pltpu.VMEM((_PAD_LEN + 16,), jnp.int32),
        pltpu.VMEM((_PAD_LEN, _HID), jnp.float32),
        pltpu.VMEM((_HID, 16), jnp.float32),
        pltpu.VMEM((_HID,), jnp.float32),
        pltpu.VMEM((_HID,), jnp.float32),
        pltpu.SemaphoreType.DMA,
    ],
    compiler_params=pltpu.CompilerParams(use_tc_tiling_on_sc=False,
                                         needs_layout_passes=False),
)


@jax.jit
def _run(tok_flat, word_table, posT_pad, gamma, beta):
    out = _emb(tok_flat, word_table, posT_pad, gamma, beta)
    return out.reshape(_BATCH, _MAXLEN, _HID)


def kernel(tokens, word_table, pos_table, gamma, beta):
    tok_flat = tokens.reshape(-1).astype(jnp.int32)
    posT_pad = jnp.pad(pos_table.T, ((0, 0), (0, _PAD_LEN - _MAXLEN)))
    return _run(tok_flat, word_table, posT_pad, gamma, beta)


# speedup vs baseline: 3.1582x; 3.1582x over previous
import jax
import jax.numpy as jnp
from jax import lax
from jax.experimental import pallas as pl
from jax.experimental.pallas import tpu as pltpu
from jax.experimental.pallas import tpu_sc as plsc

_VOCAB = 100000
_HID = 64
_MAXLEN = 200
_BATCH = 4096
_EPS = 1e-8
_NC = 2
_NS = 16
_NW = _NC * _NS
_NTOK = _BATCH * _MAXLEN
_TOK_PER_W = _NTOK // _NW
_CHUNK = 512
_NCHUNK = _TOK_PER_W // _CHUNK


def _gather_body(tok_hbm, word_hbm, rows_hbm,
                 idx0, idx1, buf0, buf1, sem0, sem1, semo):
    wid = lax.axis_index("s") * _NC + lax.axis_index("c")
    base = wid * _TOK_PER_W
    idx_v = [idx0, idx1]
    buf_v = [buf0, buf1]
    sems = [sem0, sem1]

    def issue(i, slot):
        off = base + i * _CHUNK
        pltpu.sync_copy(tok_hbm.at[pl.ds(off, _CHUNK)], idx_v[slot])
        return [pltpu.async_copy(
            word_hbm.at[idx_v[slot].at[pl.ds(128 * j, 128)]],
            buf_v[slot].at[pl.ds(128 * j, 128)], sems[slot])
            for j in range(_CHUNK // 128)]

    for c in issue(0, 0):
        pass

    def chunk_body(i, c):
        slot = lax.rem(i, 2)

        @pl.when(i + 1 < _NCHUNK)
        def _():
            off = base + (i + 1) * _CHUNK
            nslot = 1 - slot
            @pl.when(nslot == 1)
            def _():
                pltpu.sync_copy(tok_hbm.at[pl.ds(off, _CHUNK)], idx1)
                for j in range(_CHUNK // 128):
                    pltpu.async_copy(word_hbm.at[idx1.at[pl.ds(128 * j, 128)]],
                                     buf1.at[pl.ds(128 * j, 128)], sem1)

            @pl.when(nslot == 0)
            def _():
                pltpu.sync_copy(tok_hbm.at[pl.ds(off, _CHUNK)], idx0)
                for j in range(_CHUNK // 128):
                    pltpu.async_copy(word_hbm.at[idx0.at[pl.ds(128 * j, 128)]],
                                     buf0.at[pl.ds(128 * j, 128)], sem0)

        off_i = base + i * _CHUNK

        @pl.when(slot == 0)
        def _():
            for j in range(_CHUNK // 128):
                pltpu.make_async_copy(word_hbm.at[idx0.at[pl.ds(128 * j, 128)]],
                                      buf0.at[pl.ds(128 * j, 128)], sem0).wait()
            pltpu.async_copy(buf0, rows_hbm.at[pl.ds(off_i, _CHUNK)], semo).wait()

        @pl.when(slot == 1)
        def _():
            for j in range(_CHUNK // 128):
                pltpu.make_async_copy(word_hbm.at[idx1.at[pl.ds(128 * j, 128)]],
                                      buf1.at[pl.ds(128 * j, 128)], sem1).wait()
            pltpu.async_copy(buf1, rows_hbm.at[pl.ds(off_i, _CHUNK)], semo).wait()

        return c

    lax.fori_loop(0, _NCHUNK, chunk_body, 0)


_gather = pl.kernel(
    _gather_body,
    mesh=plsc.VectorSubcoreMesh(core_axis_name="c", subcore_axis_name="s"),
    out_type=jax.ShapeDtypeStruct((_NTOK, _HID), jnp.float32),
    scratch_types=[
        pltpu.VMEM((_CHUNK,), jnp.int32),
        pltpu.VMEM((_CHUNK,), jnp.int32),
        pltpu.VMEM((_CHUNK, _HID), jnp.float32),
        pltpu.VMEM((_CHUNK, _HID), jnp.float32),
        pltpu.SemaphoreType.DMA,
        pltpu.SemaphoreType.DMA,
        pltpu.SemaphoreType.DMA,
    ],
    compiler_params=pltpu.CompilerParams(use_tc_tiling_on_sc=False),
)


_SEQ_BLK = 16


def _ln_body(tok_ref, x_ref, pos_ref, gamma_ref, beta_ref, o_ref):
    x = x_ref[...]
    x = x + pos_ref[...][None, :, :]
    mean = jnp.mean(x, axis=-1, keepdims=True)
    d = x - mean
    var = jnp.mean(d * d, axis=-1, keepdims=True)
    normed = d * lax.rsqrt(var + _EPS)
    y = normed * gamma_ref[...][None, None, :] + beta_ref[...][None, None, :]
    mask = (tok_ref[...] != 0).astype(jnp.float32)[:, :, None]
    o_ref[...] = y * mask


def _ln(rows3, tokens, pos_table, gamma, beta):
    grid = (_BATCH // _SEQ_BLK,)
    return pl.pallas_call(
        _ln_body,
        grid=grid,
        in_specs=[
            pl.BlockSpec((_SEQ_BLK, _MAXLEN), lambda i: (i, 0)),
            pl.BlockSpec((_SEQ_BLK, _MAXLEN, _HID), lambda i: (i, 0, 0)),
            pl.BlockSpec((_MAXLEN, _HID), lambda i: (0, 0)),
            pl.BlockSpec((_HID,), lambda i: (0,)),
            pl.BlockSpec((_HID,), lambda i: (0,)),
        ],
        out_specs=pl.BlockSpec((_SEQ_BLK, _MAXLEN, _HID), lambda i: (i, 0, 0)),
        out_shape=jax.ShapeDtypeStruct((_BATCH, _MAXLEN, _HID), jnp.float32),
    )(tokens, rows3, pos_table, gamma, beta)


@jax.jit
def _run(tokens, tok_flat, word_table, pos_table, gamma, beta):
    rows = _gather(tok_flat, word_table)
    rows3 = rows.reshape(_BATCH, _MAXLEN, _HID)
    return _ln(rows3, tokens, pos_table, gamma, beta)


def kernel(tokens, word_table, pos_table, gamma, beta):
    tok_flat = tokens.reshape(-1).astype(jnp.int32)
    return _run(tokens, tok_flat, word_table, pos_table, gamma, beta)
